# Initial kernel scaffold; baseline (speedup 1.0000x reference)
import jax
import jax.numpy as jnp
from jax.experimental import pallas as pl


def kernel(x, W1, b1, g1, be1, W2, b2, g2, be2, W3, b3, g3, be3, Wp, bp, ln_g, ln_b):
    raise NotImplementedError("write your pallas kernel here")


# speedup vs baseline: 8.8696x; 8.8696x over previous
import functools

import jax
import jax.numpy as jnp
from jax import lax
from jax.experimental import pallas as pl
from jax.experimental.pallas import tpu as pltpu
from jax.experimental.pallas import tpu_sc as plsc

F32 = jnp.float32
NBLK = 2048
NW = 32
SC_CHUNK = 8192
N_PTS = 16384


def _p1_body(x_ref, w_ref, b_ref, su_ref, sq_ref, ymn_ref, ymx_ref):
    bi = pl.program_id(0)
    j = pl.program_id(1)
    xb = x_ref[0]
    u = jnp.dot(xb, w_ref[...], preferred_element_type=F32) + b_ref[...]
    su = jnp.sum(u, axis=0, keepdims=True)
    sq = jnp.sum(u * u, axis=0, keepdims=True)
    y = xb[:, 1:2]
    mn = jnp.reshape(jnp.min(y), (1, 1, 1))
    mx = jnp.reshape(jnp.max(y), (1, 1, 1))
    first = jnp.logical_and(bi == 0, j == 0)

    @pl.when(first)
    def _():
        su_ref[...] = su
        sq_ref[...] = sq

    @pl.when(jnp.logical_not(first))
    def _():
        su_ref[...] += su
        sq_ref[...] += sq

    @pl.when(j == 0)
    def _():
        ymn_ref[...] = mn
        ymx_ref[...] = mx

    @pl.when(j != 0)
    def _():
        ymn_ref[...] = jnp.minimum(ymn_ref[...], mn)
        ymx_ref[...] = jnp.maximum(ymx_ref[...], mx)


def _p2_body(x_ref, a1_ref, c1_ref, w_ref, b_ref, su_ref, sq_ref):
    bi = pl.program_id(0)
    j = pl.program_id(1)
    xb = x_ref[0]
    h1 = jnp.maximum(
        jnp.dot(xb, a1_ref[...], preferred_element_type=F32) + c1_ref[...], 0.0)
    u = jnp.dot(h1, w_ref[...], preferred_element_type=F32) + b_ref[...]
    su = jnp.sum(u, axis=0, keepdims=True)
    sq = jnp.sum(u * u, axis=0, keepdims=True)
    first = jnp.logical_and(bi == 0, j == 0)

    @pl.when(first)
    def _():
        su_ref[...] = su
        sq_ref[...] = sq

    @pl.when(jnp.logical_not(first))
    def _():
        su_ref[...] += su
        sq_ref[...] += sq


def _p3_body(x_ref, a1_ref, c1_ref, a2_ref, c2_ref, w_ref, b_ref,
             su_ref, sq_ref):
    bi = pl.program_id(0)
    j = pl.program_id(1)
    xb = x_ref[0]
    h1 = jnp.maximum(
        jnp.dot(xb, a1_ref[...], preferred_element_type=F32) + c1_ref[...], 0.0)
    h2 = jnp.maximum(
        jnp.dot(h1, a2_ref[...], preferred_element_type=F32) + c2_ref[...], 0.0)
    u = jnp.dot(h2, w_ref[...], preferred_element_type=F32) + b_ref[...]
    su = jnp.sum(u, axis=0, keepdims=True)
    sq = jnp.sum(u * u, axis=0, keepdims=True)
    first = jnp.logical_and(bi == 0, j == 0)

    @pl.when(first)
    def _():
        su_ref[...] = su
        sq_ref[...] = sq

    @pl.when(jnp.logical_not(first))
    def _():
        su_ref[...] += su
        sq_ref[...] += sq


def _p4_body(x_ref, a1_ref, c1_ref, a2_ref, c2_ref, a3_ref, c3_ref,
             ctr_ref, ymn_ref, den_ref,
             mxh_ref, smh_ref, mxs_ref, sms_ref):
    j = pl.program_id(1)
    xb = x_ref[0]
    h1 = jnp.maximum(
        jnp.dot(xb, a1_ref[...], preferred_element_type=F32) + c1_ref[...], 0.0)
    h2 = jnp.maximum(
        jnp.dot(h1, a2_ref[...], preferred_element_type=F32) + c2_ref[...], 0.0)
    h3 = jnp.maximum(
        jnp.dot(h2, a3_ref[...], preferred_element_type=F32) + c3_ref[...], 0.0)
    mxh = jnp.max(h3, axis=0, keepdims=True)[None]
    smh = jnp.sum(h3, axis=0, keepdims=True)[None]

    xs = xb[:, 0:1]
    yv = xb[:, 1:2]
    zs = xb[:, 2:3]
    t = (yv - ymn_ref[0]) / den_ref[0] * 10.0
    bk = t.astype(jnp.int32)
    oh = (bk == lax.broadcasted_iota(jnp.int32, (1, 16), 1)).astype(F32)
    cpt = jnp.dot(oh, ctr_ref[0], preferred_element_type=F32)
    dx = xs - cpt[:, 0:1]
    dz = zs - cpt[:, 1:2]
    r = jnp.sqrt(dx * dx + dz * dz)
    inv = jnp.where(r > 0.0, 1.0 / r, 0.0)
    sn = dz * inv
    cs = jnp.where(r > 0.0, dx * inv, 1.0)
    sp = jnp.concatenate([sn, cs, r], axis=1)
    mxs = jnp.max(sp, axis=0, keepdims=True)[None]
    sms = jnp.sum(sp, axis=0, keepdims=True)[None]

    @pl.when(j == 0)
    def _():
        mxh_ref[...] = mxh
        smh_ref[...] = smh
        mxs_ref[...] = mxs
        sms_ref[...] = sms

    @pl.when(j != 0)
    def _():
        mxh_ref[...] = jnp.maximum(mxh_ref[...], mxh)
        smh_ref[...] += smh
        mxs_ref[...] = jnp.maximum(mxs_ref[...], mxs)
        sms_ref[...] += sms


def _p5_body(mxh_ref, mxs_ref, smh_ref, sms_ref,
             w1_ref, w2_ref, w3_ref, w4_ref,
             bp_ref, g_ref, b_ref, out_ref):
    invn = 1.0 / N_PTS
    o = (jnp.dot(mxh_ref[...], w1_ref[...], preferred_element_type=F32)
         + jnp.dot(mxs_ref[...], w2_ref[...], preferred_element_type=F32)
         + jnp.dot(smh_ref[...] * invn, w3_ref[...], preferred_element_type=F32)
         + jnp.dot(sms_ref[...] * invn, w4_ref[...], preferred_element_type=F32)
         + bp_ref[...])
    mu = jnp.mean(o, axis=1, keepdims=True)
    var = jnp.mean((o - mu) ** 2, axis=1, keepdims=True)
    out_ref[...] = (o - mu) / jnp.sqrt(var + 1e-5) * g_ref[...] + b_ref[...]


def _sc_hist_body(xs_hbm, ys_hbm, zs_hbm, ymn_hbm, den_hbm, out_hbm,
                  xv, yv, zv, ymv, dnv, acc):
    c = lax.axis_index("c")
    s = lax.axis_index("s")
    w = s * 2 + c
    base = w * SC_CHUNK
    pltpu.sync_copy(xs_hbm.at[pl.ds(base, SC_CHUNK)], xv)
    pltpu.sync_copy(ys_hbm.at[pl.ds(base, SC_CHUNK)], yv)
    pltpu.sync_copy(zs_hbm.at[pl.ds(base, SC_CHUNK)], zv)
    pltpu.sync_copy(ymn_hbm.at[pl.ds(w * 16, 16)], ymv)
    pltpu.sync_copy(den_hbm.at[pl.ds(w * 16, 16)], dnv)
    ymin_v = ymv[...]
    den_v = dnv[...]
    for i in range(3):
        for r in range(16):
            acc[i, r] = jnp.zeros((16,), F32)
    lanes = lax.iota(jnp.int32, 16)
    idx0 = jnp.zeros((16,), jnp.int32)
    idx1 = idx0 + 1
    idx2 = idx0 + 2
    ones_f = jnp.ones((16,), F32)

    def body(i, carry):
        off = i * 16
        y16 = yv[pl.ds(off, 16)]
        x16 = xv[pl.ds(off, 16)]
        z16 = zv[pl.ds(off, 16)]
        t = (y16 - ymin_v) / den_v * 10.0
        bk = t.astype(jnp.int32)
        plsc.addupdate_scatter(acc, [idx0, bk, lanes], ones_f)
        plsc.addupdate_scatter(acc, [idx1, bk, lanes], x16)
        plsc.addupdate_scatter(acc, [idx2, bk, lanes], z16)
        return carry

    lax.fori_loop(0, SC_CHUNK // 16, body, 0)
    pltpu.sync_copy(acc, out_hbm.at[w])


def _sc_hist(xs_f, ys_f, zs_f, ymn_f, den_f):
    fn = functools.partial(
        pl.kernel,
        mesh=plsc.VectorSubcoreMesh(core_axis_name="c", subcore_axis_name="s"),
        out_type=jax.ShapeDtypeStruct((NW, 3, 16, 16), F32),
        compiler_params=pltpu.CompilerParams(needs_layout_passes=False),
        scratch_types=[
            pltpu.VMEM((SC_CHUNK,), F32),
            pltpu.VMEM((SC_CHUNK,), F32),
            pltpu.VMEM((SC_CHUNK,), F32),
            pltpu.VMEM((16,), F32),
            pltpu.VMEM((16,), F32),
            pltpu.VMEM((3, 16, 16), F32),
        ],
    )(_sc_hist_body)
    return fn(xs_f, ys_f, zs_f, ymn_f, den_f)


def kernel(x, W1, b1, g1, be1, W2, b2, g2, be2, W3, b3, g3, be3,
           Wp, bp, ln_g, ln_b):
    B, N, _ = x.shape
    NJ = N // NBLK
    M = B * N
    minv = 1.0 / M
    grid = (B, NJ)
    xspec = pl.BlockSpec((1, NBLK, 3), lambda b_, j: (b_, j, 0))

    def full(shp):
        return pl.BlockSpec(shp, lambda b_, j: (0,) * len(shp))

    def sds(shp):
        return jax.ShapeDtypeStruct(shp, F32)

    su1, sq1, ymn, ymx = pl.pallas_call(
        _p1_body,
        grid=grid,
        in_specs=[xspec, full((3, 64)), full((1, 64))],
        out_specs=[full((1, 64)), full((1, 64)),
                   pl.BlockSpec((1, 1, 1), lambda b_, j: (b_, 0, 0)),
                   pl.BlockSpec((1, 1, 1), lambda b_, j: (b_, 0, 0))],
        out_shape=[sds((1, 64)), sds((1, 64)),
                   sds((B, 1, 1)), sds((B, 1, 1))],
    )(x, W1.T, b1[None])

    m1 = su1 * minv
    v1 = sq1 * minv - m1 * m1
    s1 = g1[None] / jnp.sqrt(v1 + 1e-5)
    a1 = W1.T * s1
    c1 = (b1[None] - m1) * s1 + be1[None]

    den = (ymx - ymn) + 1e-6
    ymn_w = jnp.repeat(ymn.reshape(B), (NW // B) * 16)
    den_w = jnp.repeat(den.reshape(B), (NW // B) * 16)
    xs_f = x[:, :, 0].reshape(M)
    ys_f = x[:, :, 1].reshape(M)
    zs_f = x[:, :, 2].reshape(M)
    part = _sc_hist(xs_f, ys_f, zs_f, ymn_w, den_w)
    agg = part.reshape(B, NW // B, 3, 16, 16).sum(axis=(1, 4))
    cnt = agg[:, 0]
    safe = jnp.maximum(cnt, 1.0)
    pos = cnt > 0.0
    cx = jnp.where(pos, agg[:, 1] / safe, 0.0)
    cz = jnp.where(pos, agg[:, 2] / safe, 0.0)
    ctr = jnp.stack([cx, cz], axis=-1)

    su2, sq2 = pl.pallas_call(
        _p2_body,
        grid=grid,
        in_specs=[xspec, full((3, 64)), full((1, 64)),
                  full((64, 128)), full((1, 128))],
        out_specs=[full((1, 128)), full((1, 128))],
        out_shape=[sds((1, 128)), sds((1, 128))],
    )(x, a1, c1, W2.T, b2[None])

    m2 = su2 * minv
    v2 = sq2 * minv - m2 * m2
    s2 = g2[None] / jnp.sqrt(v2 + 1e-5)
    a2 = W2.T * s2
    c2 = (b2[None] - m2) * s2 + be2[None]

    su3, sq3 = pl.pallas_call(
        _p3_body,
        grid=grid,
        in_specs=[xspec, full((3, 64)), full((1, 64)),
                  full((64, 128)), full((1, 128)),
                  full((128, 256)), full((1, 256))],
        out_specs=[full((1, 256)), full((1, 256))],
        out_shape=[sds((1, 256)), sds((1, 256))],
    )(x, a1, c1, a2, c2, W3.T, b3[None])

    m3 = su3 * minv
    v3 = sq3 * minv - m3 * m3
    s3 = g3[None] / jnp.sqrt(v3 + 1e-5)
    a3 = W3.T * s3
    c3 = (b3[None] - m3) * s3 + be3[None]

    den3 = den
    mxh, smh, mxs, sms = pl.pallas_call(
        _p4_body,
        grid=grid,
        in_specs=[xspec, full((3, 64)), full((1, 64)),
                  full((64, 128)), full((1, 128)),
                  full((128, 256)), full((1, 256)),
                  pl.BlockSpec((1, 16, 2), lambda b_, j: (b_, 0, 0)),
                  pl.BlockSpec((1, 1, 1), lambda b_, j: (b_, 0, 0)),
                  pl.BlockSpec((1, 1, 1), lambda b_, j: (b_, 0, 0))],
        out_specs=[pl.BlockSpec((1, 1, 256), lambda b_, j: (b_, 0, 0)),
                   pl.BlockSpec((1, 1, 256), lambda b_, j: (b_, 0, 0)),
                   pl.BlockSpec((1, 1, 3), lambda b_, j: (b_, 0, 0)),
                   pl.BlockSpec((1, 1, 3), lambda b_, j: (b_, 0, 0))],
        out_shape=[sds((B, 1, 256)), sds((B, 1, 256)),
                   sds((B, 1, 3)), sds((B, 1, 3))],
    )(x, a1, c1, a2, c2, a3, c3, ctr, ymn, den3)

    wpt = Wp.T
    out = pl.pallas_call(
        _p5_body,
        out_shape=sds((B, 512)),
    )(mxh.reshape(B, 256), mxs.reshape(B, 3),
      smh.reshape(B, 256), sms.reshape(B, 3),
      wpt[0:256], wpt[256:259], wpt[259:515], wpt[515:518],
      bp[None], ln_g[None], ln_b[None])
    return out


# speedup vs baseline: 13.7191x; 1.5468x over previous
import functools

import jax
import jax.numpy as jnp
from jax import lax
from jax.experimental import pallas as pl
from jax.experimental.pallas import tpu as pltpu
from jax.experimental.pallas import tpu_sc as plsc

F32 = jnp.float32
NBLK = 2048
N_PTS = 16384
NBATCH = 16


def _p2_body(x_ref, a1_ref, c1_ref, w_ref, b_ref, su_ref, sq_ref):
    bi = pl.program_id(0)
    j = pl.program_id(1)
    xb = x_ref[0]
    h1 = jnp.maximum(
        jnp.dot(xb, a1_ref[...], preferred_element_type=F32) + c1_ref[...], 0.0)
    u = jnp.dot(h1, w_ref[...], preferred_element_type=F32) + b_ref[...]
    su = jnp.sum(u, axis=0, keepdims=True)
    sq = jnp.sum(u * u, axis=0, keepdims=True)
    first = jnp.logical_and(bi == 0, j == 0)

    @pl.when(first)
    def _():
        su_ref[...] = su
        sq_ref[...] = sq

    @pl.when(jnp.logical_not(first))
    def _():
        su_ref[...] += su
        sq_ref[...] += sq


def _p3_body(x_ref, a1_ref, c1_ref, a2_ref, c2_ref, w_ref, b_ref,
             su_ref, sq_ref):
    bi = pl.program_id(0)
    j = pl.program_id(1)
    xb = x_ref[0]
    h1 = jnp.maximum(
        jnp.dot(xb, a1_ref[...], preferred_element_type=F32) + c1_ref[...], 0.0)
    h2 = jnp.maximum(
        jnp.dot(h1, a2_ref[...], preferred_element_type=F32) + c2_ref[...], 0.0)
    u = jnp.dot(h2, w_ref[...], preferred_element_type=F32) + b_ref[...]
    su = jnp.sum(u, axis=0, keepdims=True)
    sq = jnp.sum(u * u, axis=0, keepdims=True)
    first = jnp.logical_and(bi == 0, j == 0)

    @pl.when(first)
    def _():
        su_ref[...] = su
        sq_ref[...] = sq

    @pl.when(jnp.logical_not(first))
    def _():
        su_ref[...] += su
        sq_ref[...] += sq


def _p4_body(x_ref, xt_ref, a1_ref, c1_ref, a2_ref, c2_ref, a3_ref, c3_ref,
             ctr_ref, prm_ref,
             mxh_ref, smh_ref, mxs_ref, sms_ref):
    j = pl.program_id(1)
    xb = x_ref[0]
    h1 = jnp.maximum(
        jnp.dot(xb, a1_ref[...], preferred_element_type=F32) + c1_ref[...], 0.0)
    h2 = jnp.maximum(
        jnp.dot(h1, a2_ref[...], preferred_element_type=F32) + c2_ref[...], 0.0)
    h3 = jnp.maximum(
        jnp.dot(h2, a3_ref[...], preferred_element_type=F32) + c3_ref[...], 0.0)
    mxh = jnp.max(h3, axis=0, keepdims=True)[None]
    smh = jnp.sum(h3, axis=0, keepdims=True)[None]

    xtb = xt_ref[0]
    xs = xtb[0:1]
    yv = xtb[1:2]
    zs = xtb[2:3]
    t = (yv - prm_ref[0, 0, 0]) / prm_ref[0, 0, 1] * 10.0
    bk = t.astype(jnp.int32)
    cx = jnp.zeros((1, NBLK), F32)
    cz = jnp.zeros((1, NBLK), F32)
    for k in range(11):
        mk = bk == k
        cx = jnp.where(mk, ctr_ref[0, 0, k], cx)
        cz = jnp.where(mk, ctr_ref[0, 0, 16 + k], cz)
    dx = xs - cx
    dz = zs - cz
    r = jnp.sqrt(dx * dx + dz * dz)
    inv = jnp.where(r > 0.0, 1.0 / r, 0.0)
    sn = dz * inv
    cs = jnp.where(r > 0.0, dx * inv, 1.0)

    def c3_(a, b, c):
        return jnp.concatenate(
            [jnp.reshape(a, (1, 1, 1)), jnp.reshape(b, (1, 1, 1)),
             jnp.reshape(c, (1, 1, 1))], axis=2)

    mxs = c3_(jnp.max(sn), jnp.max(cs), jnp.max(r))
    sms = c3_(jnp.sum(sn), jnp.sum(cs), jnp.sum(r))

    @pl.when(j == 0)
    def _():
        mxh_ref[...] = mxh
        smh_ref[...] = smh
        mxs_ref[...] = mxs
        sms_ref[...] = sms

    @pl.when(j != 0)
    def _():
        mxh_ref[...] = jnp.maximum(mxh_ref[...], mxh)
        smh_ref[...] += smh
        mxs_ref[...] = jnp.maximum(mxs_ref[...], mxs)
        sms_ref[...] += sms


def _p5_body(mxh_ref, mxs_ref, smh_ref, sms_ref,
             w1_ref, w2_ref, w3_ref, w4_ref,
             bp_ref, g_ref, b_ref, out_ref):
    invn = 1.0 / N_PTS
    o = (jnp.dot(mxh_ref[...], w1_ref[...], preferred_element_type=F32)
         + jnp.dot(mxs_ref[...], w2_ref[...], preferred_element_type=F32)
         + jnp.dot(smh_ref[...] * invn, w3_ref[...], preferred_element_type=F32)
         + jnp.dot(sms_ref[...] * invn, w4_ref[...], preferred_element_type=F32)
         + bp_ref[...])
    mu = jnp.mean(o, axis=1, keepdims=True)
    var = jnp.mean((o - mu) ** 2, axis=1, keepdims=True)
    out_ref[...] = (o - mu) / jnp.sqrt(var + 1e-5) * g_ref[...] + b_ref[...]


def _sc_body(xs_hbm, ys_hbm, zs_hbm, part_hbm, st_hbm, xv, yv, zv, acc, stv):
    c = lax.axis_index("c")
    s = lax.axis_index("s")
    w = s * 2 + c

    @pl.when(w < NBATCH)
    def _():
        base = w * N_PTS
        pltpu.sync_copy(xs_hbm.at[pl.ds(base, N_PTS)], xv)
        pltpu.sync_copy(ys_hbm.at[pl.ds(base, N_PTS)], yv)
        pltpu.sync_copy(zs_hbm.at[pl.ds(base, N_PTS)], zv)
        nit = N_PTS // 16
        big = jnp.float32(3.4e38)
        zero = jnp.zeros((16,), F32)
        init = (jnp.full((16,), big, F32), jnp.full((16,), -big, F32),
                zero, zero, zero, zero, zero, zero, zero, zero, zero)

        def body1(i, cr):
            mn, mx, sx, sy, sz, sxx, sxy, sxz, syy, syz, szz = cr
            off = i * 16
            xw = xv[pl.ds(off, 16)]
            yw = yv[pl.ds(off, 16)]
            zw = zv[pl.ds(off, 16)]
            return (jnp.minimum(mn, yw), jnp.maximum(mx, yw),
                    sx + xw, sy + yw, sz + zw,
                    sxx + xw * xw, sxy + xw * yw, sxz + xw * zw,
                    syy + yw * yw, syz + yw * zw, szz + zw * zw)

        st = lax.fori_loop(0, nit, body1, init)
        for i in range(11):
            stv[i] = st[i]
        stv[11] = zero
        pltpu.sync_copy(stv, st_hbm.at[w])

        mn_s = jnp.min(st[0])
        mx_s = jnp.max(st[1])
        den_s = mx_s - mn_s + 1e-6
        mn_v = jnp.full((16,), mn_s, F32)
        den_v = jnp.full((16,), den_s, F32)

        for i in range(3):
            for rr in range(16):
                acc[i, rr] = zero
        lanes = lax.iota(jnp.int32, 16)
        idx0 = jnp.zeros((16,), jnp.int32)
        idx1 = idx0 + 1
        idx2 = idx0 + 2
        ones_f = jnp.ones((16,), F32)

        def body2(i, carry):
            off = i * 16
            y16 = yv[pl.ds(off, 16)]
            x16 = xv[pl.ds(off, 16)]
            z16 = zv[pl.ds(off, 16)]
            t = (y16 - mn_v) / den_v * 10.0
            bk = t.astype(jnp.int32)
            plsc.addupdate_scatter(acc, [idx0, bk, lanes], ones_f)
            plsc.addupdate_scatter(acc, [idx1, bk, lanes], x16)
            plsc.addupdate_scatter(acc, [idx2, bk, lanes], z16)
            return carry

        lax.fori_loop(0, nit, body2, 0)
        pltpu.sync_copy(acc, part_hbm.at[w])


def _sc_hist(xs_f, ys_f, zs_f):
    fn = functools.partial(
        pl.kernel,
        mesh=plsc.VectorSubcoreMesh(core_axis_name="c", subcore_axis_name="s"),
        out_type=[jax.ShapeDtypeStruct((NBATCH, 3, 16, 16), F32),
                  jax.ShapeDtypeStruct((NBATCH, 12, 16), F32)],
        compiler_params=pltpu.CompilerParams(needs_layout_passes=False),
        scratch_types=[
            pltpu.VMEM((N_PTS,), F32),
            pltpu.VMEM((N_PTS,), F32),
            pltpu.VMEM((N_PTS,), F32),
            pltpu.VMEM((3, 16, 16), F32),
            pltpu.VMEM((12, 16), F32),
        ],
    )(_sc_body)
    return fn(xs_f, ys_f, zs_f)


def kernel(x, W1, b1, g1, be1, W2, b2, g2, be2, W3, b3, g3, be3,
           Wp, bp, ln_g, ln_b):
    B, N, _ = x.shape
    NJ = N // NBLK
    M = B * N
    minv = 1.0 / M
    grid = (B, NJ)
    xspec = pl.BlockSpec((1, NBLK, 3), lambda b_, j: (b_, j, 0))

    def full(shp):
        return pl.BlockSpec(shp, lambda b_, j: (0,) * len(shp))

    def sds(shp):
        return jax.ShapeDtypeStruct(shp, F32)

    xs_f = x[:, :, 0].reshape(M)
    ys_f = x[:, :, 1].reshape(M)
    zs_f = x[:, :, 2].reshape(M)
    part, st = _sc_hist(xs_f, ys_f, zs_f)

    ymn_b = jnp.min(st[:, 0, :], axis=1)
    ymx_b = jnp.max(st[:, 1, :], axis=1)
    den_b = ymx_b - ymn_b + 1e-6
    mom = jnp.sum(st[:, 2:11, :], axis=(0, 2))
    mu3 = mom[0:3] * minv
    cov = (jnp.stack([
        jnp.stack([mom[3], mom[4], mom[5]]),
        jnp.stack([mom[4], mom[6], mom[7]]),
        jnp.stack([mom[5], mom[7], mom[8]]),
    ]) * minv - mu3[:, None] * mu3[None, :])
    m1 = mu3 @ W1.T + b1
    v1 = jnp.einsum("jc,cd,jd->j", W1, cov, W1)
    s1 = g1 / jnp.sqrt(v1 + 1e-5)
    a1 = W1.T * s1[None]
    c1 = ((b1 - m1) * s1 + be1)[None]

    agg = jnp.sum(part, axis=3)
    cnt = agg[:, 0]
    safe = jnp.maximum(cnt, 1.0)
    pos = cnt > 0.0
    cx = jnp.where(pos, agg[:, 1] / safe, 0.0)
    cz = jnp.where(pos, agg[:, 2] / safe, 0.0)
    ctr = jnp.concatenate([cx, cz], axis=1).reshape(B, 1, 32)
    prm = jnp.stack([ymn_b, den_b], axis=1).reshape(B, 1, 2)

    su2, sq2 = pl.pallas_call(
        _p2_body,
        grid=grid,
        in_specs=[xspec, full((3, 64)), full((1, 64)),
                  full((64, 128)), full((1, 128))],
        out_specs=[full((1, 128)), full((1, 128))],
        out_shape=[sds((1, 128)), sds((1, 128))],
    )(x, a1, c1, W2.T, b2[None])

    m2 = su2 * minv
    v2 = sq2 * minv - m2 * m2
    s2 = g2[None] / jnp.sqrt(v2 + 1e-5)
    a2 = W2.T * s2
    c2 = (b2[None] - m2) * s2 + be2[None]

    su3, sq3 = pl.pallas_call(
        _p3_body,
        grid=grid,
        in_specs=[xspec, full((3, 64)), full((1, 64)),
                  full((64, 128)), full((1, 128)),
                  full((128, 256)), full((1, 256))],
        out_specs=[full((1, 256)), full((1, 256))],
        out_shape=[sds((1, 256)), sds((1, 256))],
    )(x, a1, c1, a2, c2, W3.T, b3[None])

    m3 = su3 * minv
    v3 = sq3 * minv - m3 * m3
    s3 = g3[None] / jnp.sqrt(v3 + 1e-5)
    a3 = W3.T * s3
    c3 = (b3[None] - m3) * s3 + be3[None]

    xt = x.transpose(0, 2, 1)
    mxh, smh, mxs, sms = pl.pallas_call(
        _p4_body,
        grid=grid,
        in_specs=[xspec,
                  pl.BlockSpec((1, 3, NBLK), lambda b_, j: (b_, 0, j)),
                  full((3, 64)), full((1, 64)),
                  full((64, 128)), full((1, 128)),
                  full((128, 256)), full((1, 256)),
                  pl.BlockSpec((1, 1, 32), lambda b_, j: (b_, 0, 0),
                               memory_space=pltpu.SMEM),
                  pl.BlockSpec((1, 1, 2), lambda b_, j: (b_, 0, 0),
                               memory_space=pltpu.SMEM)],
        out_specs=[pl.BlockSpec((1, 1, 256), lambda b_, j: (b_, 0, 0)),
                   pl.BlockSpec((1, 1, 256), lambda b_, j: (b_, 0, 0)),
                   pl.BlockSpec((1, 1, 3), lambda b_, j: (b_, 0, 0)),
                   pl.BlockSpec((1, 1, 3), lambda b_, j: (b_, 0, 0))],
        out_shape=[sds((B, 1, 256)), sds((B, 1, 256)),
                   sds((B, 1, 3)), sds((B, 1, 3))],
    )(x, xt, a1, c1, a2, c2, a3, c3, ctr, prm)

    wpt = Wp.T
    out = pl.pallas_call(
        _p5_body,
        out_shape=sds((B, 512)),
    )(mxh.reshape(B, 256), mxs.reshape(B, 3),
      smh.reshape(B, 256), sms.reshape(B, 3),
      wpt[0:256], wpt[256:259], wpt[259:515], wpt[515:518],
      bp[None], ln_g[None], ln_b[None])
    return out


# speedup vs baseline: 16.7747x; 1.2227x over previous
import functools

import jax
import jax.numpy as jnp
from jax import lax
from jax.experimental import pallas as pl
from jax.experimental.pallas import tpu as pltpu
from jax.experimental.pallas import tpu_sc as plsc

F32 = jnp.float32
NBLK = 4096
N_PTS = 16384
NBATCH = 16


def _p2_body(x_ref, a1_ref, c1_ref, w_ref, b_ref, su_ref, sq_ref):
    bi = pl.program_id(0)
    j = pl.program_id(1)
    xb = x_ref[0]
    h1 = jnp.maximum(
        jnp.dot(xb, a1_ref[...], preferred_element_type=F32) + c1_ref[...], 0.0)
    u = jnp.dot(h1, w_ref[...], preferred_element_type=F32) + b_ref[...]
    su = jnp.sum(u, axis=0, keepdims=True)
    sq = jnp.sum(u * u, axis=0, keepdims=True)
    first = jnp.logical_and(bi == 0, j == 0)

    @pl.when(first)
    def _():
        su_ref[...] = su
        sq_ref[...] = sq

    @pl.when(jnp.logical_not(first))
    def _():
        su_ref[...] += su
        sq_ref[...] += sq


def _p3_body(x_ref, a1_ref, c1_ref, a2_ref, c2_ref, w_ref, b_ref,
             su_ref, sq_ref):
    bi = pl.program_id(0)
    j = pl.program_id(1)
    xb = x_ref[0]
    h1 = jnp.maximum(
        jnp.dot(xb, a1_ref[...], preferred_element_type=F32) + c1_ref[...], 0.0)
    h2 = jnp.maximum(
        jnp.dot(h1, a2_ref[...], preferred_element_type=F32) + c2_ref[...], 0.0)
    u = jnp.dot(h2, w_ref[...], preferred_element_type=F32) + b_ref[...]
    su = jnp.sum(u, axis=0, keepdims=True)
    sq = jnp.sum(u * u, axis=0, keepdims=True)
    first = jnp.logical_and(bi == 0, j == 0)

    @pl.when(first)
    def _():
        su_ref[...] = su
        sq_ref[...] = sq

    @pl.when(jnp.logical_not(first))
    def _():
        su_ref[...] += su
        sq_ref[...] += sq


def _p4_body(x_ref, xt_ref, a1_ref, c1_ref, a2_ref, c2_ref, a3_ref, c3_ref,
             ctr_ref, prm_ref,
             mxh_ref, smh_ref, mxs_ref, sms_ref):
    j = pl.program_id(1)
    xb = x_ref[0]
    h1 = jnp.maximum(
        jnp.dot(xb, a1_ref[...], preferred_element_type=F32) + c1_ref[...], 0.0)
    h2 = jnp.maximum(
        jnp.dot(h1, a2_ref[...], preferred_element_type=F32) + c2_ref[...], 0.0)
    h3 = jnp.maximum(
        jnp.dot(h2, a3_ref[...], preferred_element_type=F32) + c3_ref[...], 0.0)
    mxh = jnp.max(h3, axis=0, keepdims=True)[None]
    smh = jnp.sum(h3, axis=0, keepdims=True)[None]

    xtb = xt_ref[0]
    xs = xtb[0:1]
    yv = xtb[1:2]
    zs = xtb[2:3]
    t = (yv - prm_ref[0, 0, 0]) / prm_ref[0, 0, 1] * 10.0
    bk = t.astype(jnp.int32)
    cx = jnp.zeros((1, NBLK), F32)
    cz = jnp.zeros((1, NBLK), F32)
    for k in range(11):
        mk = bk == k
        cx = jnp.where(mk, ctr_ref[0, 0, k], cx)
        cz = jnp.where(mk, ctr_ref[0, 0, 16 + k], cz)
    dx = xs - cx
    dz = zs - cz
    r = jnp.sqrt(dx * dx + dz * dz)
    inv = jnp.where(r > 0.0, 1.0 / r, 0.0)
    sn = dz * inv
    cs = jnp.where(r > 0.0, dx * inv, 1.0)

    def c3_(a, b, c):
        return jnp.concatenate(
            [jnp.reshape(a, (1, 1, 1)), jnp.reshape(b, (1, 1, 1)),
             jnp.reshape(c, (1, 1, 1))], axis=2)

    mxs = c3_(jnp.max(sn), jnp.max(cs), jnp.max(r))
    sms = c3_(jnp.sum(sn), jnp.sum(cs), jnp.sum(r))

    @pl.when(j == 0)
    def _():
        mxh_ref[...] = mxh
        smh_ref[...] = smh
        mxs_ref[...] = mxs
        sms_ref[...] = sms

    @pl.when(j != 0)
    def _():
        mxh_ref[...] = jnp.maximum(mxh_ref[...], mxh)
        smh_ref[...] += smh
        mxs_ref[...] = jnp.maximum(mxs_ref[...], mxs)
        sms_ref[...] += sms


def _p5_body(mxh_ref, mxs_ref, smh_ref, sms_ref,
             w1_ref, w2_ref, w3_ref, w4_ref,
             bp_ref, g_ref, b_ref, out_ref):
    invn = 1.0 / N_PTS
    o = (jnp.dot(mxh_ref[...], w1_ref[...], preferred_element_type=F32)
         + jnp.dot(mxs_ref[...], w2_ref[...], preferred_element_type=F32)
         + jnp.dot(smh_ref[...] * invn, w3_ref[...], preferred_element_type=F32)
         + jnp.dot(sms_ref[...] * invn, w4_ref[...], preferred_element_type=F32)
         + bp_ref[...])
    mu = jnp.mean(o, axis=1, keepdims=True)
    var = jnp.mean((o - mu) ** 2, axis=1, keepdims=True)
    out_ref[...] = (o - mu) / jnp.sqrt(var + 1e-5) * g_ref[...] + b_ref[...]


def _sc_body(xs_hbm, ys_hbm, zs_hbm, part_hbm, st_hbm, xv, yv, zv, acc, stv):
    c = lax.axis_index("c")
    s = lax.axis_index("s")
    w = s * 2 + c

    @pl.when(w < NBATCH)
    def _():
        base = w * N_PTS
        pltpu.sync_copy(xs_hbm.at[pl.ds(base, N_PTS)], xv)
        pltpu.sync_copy(ys_hbm.at[pl.ds(base, N_PTS)], yv)
        pltpu.sync_copy(zs_hbm.at[pl.ds(base, N_PTS)], zv)
        nit = N_PTS // 16
        big = jnp.float32(3.4e38)
        zero = jnp.zeros((16,), F32)
        init = (jnp.full((16,), big, F32), jnp.full((16,), -big, F32),
                zero, zero, zero, zero, zero, zero, zero, zero, zero)

        def body1(i, cr):
            mn, mx, sx, sy, sz, sxx, sxy, sxz, syy, syz, szz = cr
            off = i * 16
            xw = xv[pl.ds(off, 16)]
            yw = yv[pl.ds(off, 16)]
            zw = zv[pl.ds(off, 16)]
            return (jnp.minimum(mn, yw), jnp.maximum(mx, yw),
                    sx + xw, sy + yw, sz + zw,
                    sxx + xw * xw, sxy + xw * yw, sxz + xw * zw,
                    syy + yw * yw, syz + yw * zw, szz + zw * zw)

        st = lax.fori_loop(0, nit, body1, init)
        for i in range(11):
            stv[i] = st[i]
        stv[11] = zero
        pltpu.sync_copy(stv, st_hbm.at[w])

        mn_s = jnp.min(st[0])
        mx_s = jnp.max(st[1])
        den_s = mx_s - mn_s + 1e-6
        mn_v = jnp.full((16,), mn_s, F32)
        den_v = jnp.full((16,), den_s, F32)

        for i in range(3):
            for rr in range(16):
                acc[i, rr] = zero
        lanes = lax.iota(jnp.int32, 16)
        idx0 = jnp.zeros((16,), jnp.int32)
        idx1 = idx0 + 1
        idx2 = idx0 + 2
        ones_f = jnp.ones((16,), F32)

        def body2(i, carry):
            off = i * 16
            y16 = yv[pl.ds(off, 16)]
            x16 = xv[pl.ds(off, 16)]
            z16 = zv[pl.ds(off, 16)]
            t = (y16 - mn_v) / den_v * 10.0
            bk = t.astype(jnp.int32)
            plsc.addupdate_scatter(acc, [idx0, bk, lanes], ones_f)
            plsc.addupdate_scatter(acc, [idx1, bk, lanes], x16)
            plsc.addupdate_scatter(acc, [idx2, bk, lanes], z16)
            return carry

        lax.fori_loop(0, nit, body2, 0)
        pltpu.sync_copy(acc, part_hbm.at[w])


def _sc_hist(xs_f, ys_f, zs_f):
    fn = functools.partial(
        pl.kernel,
        mesh=plsc.VectorSubcoreMesh(core_axis_name="c", subcore_axis_name="s"),
        out_type=[jax.ShapeDtypeStruct((NBATCH, 3, 16, 16), F32),
                  jax.ShapeDtypeStruct((NBATCH, 12, 16), F32)],
        compiler_params=pltpu.CompilerParams(needs_layout_passes=False),
        scratch_types=[
            pltpu.VMEM((N_PTS,), F32),
            pltpu.VMEM((N_PTS,), F32),
            pltpu.VMEM((N_PTS,), F32),
            pltpu.VMEM((3, 16, 16), F32),
            pltpu.VMEM((12, 16), F32),
        ],
    )(_sc_body)
    return fn(xs_f, ys_f, zs_f)


def kernel(x, W1, b1, g1, be1, W2, b2, g2, be2, W3, b3, g3, be3,
           Wp, bp, ln_g, ln_b):
    B, N, _ = x.shape
    NJ = N // NBLK
    M = B * N
    minv = 1.0 / M
    grid = (B, NJ)
    xspec = pl.BlockSpec((1, NBLK, 3), lambda b_, j: (b_, j, 0))

    def full(shp):
        return pl.BlockSpec(shp, lambda b_, j: (0,) * len(shp))

    def sds(shp):
        return jax.ShapeDtypeStruct(shp, F32)

    xs_f = x[:, :, 0].reshape(M)
    ys_f = x[:, :, 1].reshape(M)
    zs_f = x[:, :, 2].reshape(M)
    part, st = _sc_hist(xs_f, ys_f, zs_f)

    ymn_b = jnp.min(st[:, 0, :], axis=1)
    ymx_b = jnp.max(st[:, 1, :], axis=1)
    den_b = ymx_b - ymn_b + 1e-6
    mom = jnp.sum(st[:, 2:11, :], axis=(0, 2))
    mu3 = mom[0:3] * minv
    cov = (jnp.stack([
        jnp.stack([mom[3], mom[4], mom[5]]),
        jnp.stack([mom[4], mom[6], mom[7]]),
        jnp.stack([mom[5], mom[7], mom[8]]),
    ]) * minv - mu3[:, None] * mu3[None, :])
    m1 = mu3 @ W1.T + b1
    v1 = jnp.einsum("jc,cd,jd->j", W1, cov, W1)
    s1 = g1 / jnp.sqrt(v1 + 1e-5)
    a1 = W1.T * s1[None]
    c1 = ((b1 - m1) * s1 + be1)[None]

    agg = jnp.sum(part, axis=3)
    cnt = agg[:, 0]
    safe = jnp.maximum(cnt, 1.0)
    pos = cnt > 0.0
    cx = jnp.where(pos, agg[:, 1] / safe, 0.0)
    cz = jnp.where(pos, agg[:, 2] / safe, 0.0)
    ctr = jnp.concatenate([cx, cz], axis=1).reshape(B, 1, 32)
    prm = jnp.stack([ymn_b, den_b], axis=1).reshape(B, 1, 2)

    su2, sq2 = pl.pallas_call(
        _p2_body,
        grid=grid,
        in_specs=[xspec, full((3, 64)), full((1, 64)),
                  full((64, 128)), full((1, 128))],
        out_specs=[full((1, 128)), full((1, 128))],
        out_shape=[sds((1, 128)), sds((1, 128))],
    )(x, a1, c1, W2.T, b2[None])

    m2 = su2 * minv
    v2 = sq2 * minv - m2 * m2
    s2 = g2[None] / jnp.sqrt(v2 + 1e-5)
    a2 = W2.T * s2
    c2 = (b2[None] - m2) * s2 + be2[None]

    su3, sq3 = pl.pallas_call(
        _p3_body,
        grid=grid,
        in_specs=[xspec, full((3, 64)), full((1, 64)),
                  full((64, 128)), full((1, 128)),
                  full((128, 256)), full((1, 256))],
        out_specs=[full((1, 256)), full((1, 256))],
        out_shape=[sds((1, 256)), sds((1, 256))],
    )(x, a1, c1, a2, c2, W3.T, b3[None])

    m3 = su3 * minv
    v3 = sq3 * minv - m3 * m3
    s3 = g3[None] / jnp.sqrt(v3 + 1e-5)
    a3 = W3.T * s3
    c3 = (b3[None] - m3) * s3 + be3[None]

    xt = x.transpose(0, 2, 1)
    mxh, smh, mxs, sms = pl.pallas_call(
        _p4_body,
        grid=grid,
        in_specs=[xspec,
                  pl.BlockSpec((1, 3, NBLK), lambda b_, j: (b_, 0, j)),
                  full((3, 64)), full((1, 64)),
                  full((64, 128)), full((1, 128)),
                  full((128, 256)), full((1, 256)),
                  pl.BlockSpec((1, 1, 32), lambda b_, j: (b_, 0, 0),
                               memory_space=pltpu.SMEM),
                  pl.BlockSpec((1, 1, 2), lambda b_, j: (b_, 0, 0),
                               memory_space=pltpu.SMEM)],
        out_specs=[pl.BlockSpec((1, 1, 256), lambda b_, j: (b_, 0, 0)),
                   pl.BlockSpec((1, 1, 256), lambda b_, j: (b_, 0, 0)),
                   pl.BlockSpec((1, 1, 3), lambda b_, j: (b_, 0, 0)),
                   pl.BlockSpec((1, 1, 3), lambda b_, j: (b_, 0, 0))],
        out_shape=[sds((B, 1, 256)), sds((B, 1, 256)),
                   sds((B, 1, 3)), sds((B, 1, 3))],
    )(x, xt, a1, c1, a2, c2, a3, c3, ctr, prm)

    wpt = Wp.T
    out = pl.pallas_call(
        _p5_body,
        out_shape=sds((B, 512)),
    )(mxh.reshape(B, 256), mxs.reshape(B, 3),
      smh.reshape(B, 256), sms.reshape(B, 3),
      wpt[0:256], wpt[256:259], wpt[259:515], wpt[515:518],
      bp[None], ln_g[None], ln_b[None])
    return out


# speedup vs baseline: 17.9824x; 1.0720x over previous
import functools

import jax
import jax.numpy as jnp
from jax import lax
from jax.experimental import pallas as pl
from jax.experimental.pallas import tpu as pltpu
from jax.experimental.pallas import tpu_sc as plsc

F32 = jnp.float32
NBLK = 8192
N_PTS = 16384
NBATCH = 16


def _p2_body(x_ref, a1_ref, c1_ref, w_ref, b_ref, su_ref, sq_ref):
    bi = pl.program_id(0)
    j = pl.program_id(1)
    xb = x_ref[0]
    h1 = jnp.maximum(
        jnp.dot(xb, a1_ref[...], preferred_element_type=F32) + c1_ref[...], 0.0)
    u = jnp.dot(h1, w_ref[...], preferred_element_type=F32) + b_ref[...]
    su = jnp.sum(u, axis=0, keepdims=True)
    sq = jnp.sum(u * u, axis=0, keepdims=True)
    first = jnp.logical_and(bi == 0, j == 0)

    @pl.when(first)
    def _():
        su_ref[...] = su
        sq_ref[...] = sq

    @pl.when(jnp.logical_not(first))
    def _():
        su_ref[...] += su
        sq_ref[...] += sq


def _p3_body(x_ref, a1_ref, c1_ref, a2_ref, c2_ref, w_ref, b_ref,
             su_ref, sq_ref):
    bi = pl.program_id(0)
    j = pl.program_id(1)
    xb = x_ref[0]
    h1 = jnp.maximum(
        jnp.dot(xb, a1_ref[...], preferred_element_type=F32) + c1_ref[...], 0.0)
    h2 = jnp.maximum(
        jnp.dot(h1, a2_ref[...], preferred_element_type=F32) + c2_ref[...], 0.0)
    u = jnp.dot(h2, w_ref[...], preferred_element_type=F32) + b_ref[...]
    su = jnp.sum(u, axis=0, keepdims=True)
    sq = jnp.sum(u * u, axis=0, keepdims=True)
    first = jnp.logical_and(bi == 0, j == 0)

    @pl.when(first)
    def _():
        su_ref[...] = su
        sq_ref[...] = sq

    @pl.when(jnp.logical_not(first))
    def _():
        su_ref[...] += su
        sq_ref[...] += sq


def _p4_body(x_ref, xt_ref, a1_ref, c1_ref, a2_ref, c2_ref, a3_ref, c3_ref,
             ctr_ref, prm_ref,
             mxh_ref, smh_ref, mxs_ref, sms_ref):
    j = pl.program_id(1)
    xb = x_ref[0]
    h1 = jnp.maximum(
        jnp.dot(xb, a1_ref[...], preferred_element_type=F32) + c1_ref[...], 0.0)
    h2 = jnp.maximum(
        jnp.dot(h1, a2_ref[...], preferred_element_type=F32) + c2_ref[...], 0.0)
    h3 = jnp.maximum(
        jnp.dot(h2, a3_ref[...], preferred_element_type=F32) + c3_ref[...], 0.0)
    mxh = jnp.max(h3, axis=0, keepdims=True)[None]
    smh = jnp.sum(h3, axis=0, keepdims=True)[None]

    xtb = xt_ref[0]
    xs = xtb[0:1]
    yv = xtb[1:2]
    zs = xtb[2:3]
    t = (yv - prm_ref[0, 0, 0]) / prm_ref[0, 0, 1] * 10.0
    bk = t.astype(jnp.int32)
    cx = jnp.zeros((1, NBLK), F32)
    cz = jnp.zeros((1, NBLK), F32)
    for k in range(11):
        mk = bk == k
        cx = jnp.where(mk, ctr_ref[0, 0, k], cx)
        cz = jnp.where(mk, ctr_ref[0, 0, 16 + k], cz)
    dx = xs - cx
    dz = zs - cz
    r = jnp.sqrt(dx * dx + dz * dz)
    inv = jnp.where(r > 0.0, 1.0 / r, 0.0)
    sn = dz * inv
    cs = jnp.where(r > 0.0, dx * inv, 1.0)

    def c3_(a, b, c):
        return jnp.concatenate(
            [jnp.reshape(a, (1, 1, 1)), jnp.reshape(b, (1, 1, 1)),
             jnp.reshape(c, (1, 1, 1))], axis=2)

    mxs = c3_(jnp.max(sn), jnp.max(cs), jnp.max(r))
    sms = c3_(jnp.sum(sn), jnp.sum(cs), jnp.sum(r))

    @pl.when(j == 0)
    def _():
        mxh_ref[...] = mxh
        smh_ref[...] = smh
        mxs_ref[...] = mxs
        sms_ref[...] = sms

    @pl.when(j != 0)
    def _():
        mxh_ref[...] = jnp.maximum(mxh_ref[...], mxh)
        smh_ref[...] += smh
        mxs_ref[...] = jnp.maximum(mxs_ref[...], mxs)
        sms_ref[...] += sms


def _p5_body(mxh_ref, mxs_ref, smh_ref, sms_ref,
             w1_ref, w2_ref, w3_ref, w4_ref,
             bp_ref, g_ref, b_ref, out_ref):
    invn = 1.0 / N_PTS
    o = (jnp.dot(mxh_ref[...], w1_ref[...], preferred_element_type=F32)
         + jnp.dot(mxs_ref[...], w2_ref[...], preferred_element_type=F32)
         + jnp.dot(smh_ref[...] * invn, w3_ref[...], preferred_element_type=F32)
         + jnp.dot(sms_ref[...] * invn, w4_ref[...], preferred_element_type=F32)
         + bp_ref[...])
    mu = jnp.mean(o, axis=1, keepdims=True)
    var = jnp.mean((o - mu) ** 2, axis=1, keepdims=True)
    out_ref[...] = (o - mu) / jnp.sqrt(var + 1e-5) * g_ref[...] + b_ref[...]


def _sc_body(xs_hbm, ys_hbm, zs_hbm, part_hbm, st_hbm, xv, yv, zv, acc, stv):
    c = lax.axis_index("c")
    s = lax.axis_index("s")
    w = s * 2 + c

    @pl.when(w < NBATCH)
    def _():
        base = w * N_PTS
        pltpu.sync_copy(xs_hbm.at[pl.ds(base, N_PTS)], xv)
        pltpu.sync_copy(ys_hbm.at[pl.ds(base, N_PTS)], yv)
        pltpu.sync_copy(zs_hbm.at[pl.ds(base, N_PTS)], zv)
        nit = N_PTS // 16
        big = jnp.float32(3.4e38)
        zero = jnp.zeros((16,), F32)
        init = (jnp.full((16,), big, F32), jnp.full((16,), -big, F32),
                zero, zero, zero, zero, zero, zero, zero, zero, zero)

        def body1(i, cr):
            mn, mx, sx, sy, sz, sxx, sxy, sxz, syy, syz, szz = cr
            off = i * 16
            xw = xv[pl.ds(off, 16)]
            yw = yv[pl.ds(off, 16)]
            zw = zv[pl.ds(off, 16)]
            return (jnp.minimum(mn, yw), jnp.maximum(mx, yw),
                    sx + xw, sy + yw, sz + zw,
                    sxx + xw * xw, sxy + xw * yw, sxz + xw * zw,
                    syy + yw * yw, syz + yw * zw, szz + zw * zw)

        st = lax.fori_loop(0, nit, body1, init)
        for i in range(11):
            stv[i] = st[i]
        stv[11] = zero
        pltpu.sync_copy(stv, st_hbm.at[w])

        mn_s = jnp.min(st[0])
        mx_s = jnp.max(st[1])
        den_s = mx_s - mn_s + 1e-6
        mn_v = jnp.full((16,), mn_s, F32)
        den_v = jnp.full((16,), den_s, F32)

        for i in range(3):
            for rr in range(16):
                acc[i, rr] = zero
        lanes = lax.iota(jnp.int32, 16)
        idx0 = jnp.zeros((16,), jnp.int32)
        idx1 = idx0 + 1
        idx2 = idx0 + 2
        ones_f = jnp.ones((16,), F32)

        def body2(i, carry):
            off = i * 16
            y16 = yv[pl.ds(off, 16)]
            x16 = xv[pl.ds(off, 16)]
            z16 = zv[pl.ds(off, 16)]
            t = (y16 - mn_v) / den_v * 10.0
            bk = t.astype(jnp.int32)
            plsc.addupdate_scatter(acc, [idx0, bk, lanes], ones_f)
            plsc.addupdate_scatter(acc, [idx1, bk, lanes], x16)
            plsc.addupdate_scatter(acc, [idx2, bk, lanes], z16)
            return carry

        lax.fori_loop(0, nit, body2, 0)
        pltpu.sync_copy(acc, part_hbm.at[w])


def _sc_hist(xs_f, ys_f, zs_f):
    fn = functools.partial(
        pl.kernel,
        mesh=plsc.VectorSubcoreMesh(core_axis_name="c", subcore_axis_name="s"),
        out_type=[jax.ShapeDtypeStruct((NBATCH, 3, 16, 16), F32),
                  jax.ShapeDtypeStruct((NBATCH, 12, 16), F32)],
        compiler_params=pltpu.CompilerParams(needs_layout_passes=False),
        scratch_types=[
            pltpu.VMEM((N_PTS,), F32),
            pltpu.VMEM((N_PTS,), F32),
            pltpu.VMEM((N_PTS,), F32),
            pltpu.VMEM((3, 16, 16), F32),
            pltpu.VMEM((12, 16), F32),
        ],
    )(_sc_body)
    return fn(xs_f, ys_f, zs_f)


def kernel(x, W1, b1, g1, be1, W2, b2, g2, be2, W3, b3, g3, be3,
           Wp, bp, ln_g, ln_b):
    B, N, _ = x.shape
    NJ = N // NBLK
    M = B * N
    minv = 1.0 / M
    grid = (B, NJ)
    xspec = pl.BlockSpec((1, NBLK, 3), lambda b_, j: (b_, j, 0))

    def full(shp):
        return pl.BlockSpec(shp, lambda b_, j: (0,) * len(shp))

    def sds(shp):
        return jax.ShapeDtypeStruct(shp, F32)

    xs_f = x[:, :, 0].reshape(M)
    ys_f = x[:, :, 1].reshape(M)
    zs_f = x[:, :, 2].reshape(M)
    part, st = _sc_hist(xs_f, ys_f, zs_f)

    ymn_b = jnp.min(st[:, 0, :], axis=1)
    ymx_b = jnp.max(st[:, 1, :], axis=1)
    den_b = ymx_b - ymn_b + 1e-6
    mom = jnp.sum(st[:, 2:11, :], axis=(0, 2))
    mu3 = mom[0:3] * minv
    cov = (jnp.stack([
        jnp.stack([mom[3], mom[4], mom[5]]),
        jnp.stack([mom[4], mom[6], mom[7]]),
        jnp.stack([mom[5], mom[7], mom[8]]),
    ]) * minv - mu3[:, None] * mu3[None, :])
    m1 = mu3 @ W1.T + b1
    v1 = jnp.einsum("jc,cd,jd->j", W1, cov, W1)
    s1 = g1 / jnp.sqrt(v1 + 1e-5)
    a1 = W1.T * s1[None]
    c1 = ((b1 - m1) * s1 + be1)[None]

    agg = jnp.sum(part, axis=3)
    cnt = agg[:, 0]
    safe = jnp.maximum(cnt, 1.0)
    pos = cnt > 0.0
    cx = jnp.where(pos, agg[:, 1] / safe, 0.0)
    cz = jnp.where(pos, agg[:, 2] / safe, 0.0)
    ctr = jnp.concatenate([cx, cz], axis=1).reshape(B, 1, 32)
    prm = jnp.stack([ymn_b, den_b], axis=1).reshape(B, 1, 2)

    su2, sq2 = pl.pallas_call(
        _p2_body,
        grid=grid,
        in_specs=[xspec, full((3, 64)), full((1, 64)),
                  full((64, 128)), full((1, 128))],
        out_specs=[full((1, 128)), full((1, 128))],
        out_shape=[sds((1, 128)), sds((1, 128))],
    )(x, a1, c1, W2.T, b2[None])

    m2 = su2 * minv
    v2 = sq2 * minv - m2 * m2
    s2 = g2[None] / jnp.sqrt(v2 + 1e-5)
    a2 = W2.T * s2
    c2 = (b2[None] - m2) * s2 + be2[None]

    su3, sq3 = pl.pallas_call(
        _p3_body,
        grid=grid,
        in_specs=[xspec, full((3, 64)), full((1, 64)),
                  full((64, 128)), full((1, 128)),
                  full((128, 256)), full((1, 256))],
        out_specs=[full((1, 256)), full((1, 256))],
        out_shape=[sds((1, 256)), sds((1, 256))],
    )(x, a1, c1, a2, c2, W3.T, b3[None])

    m3 = su3 * minv
    v3 = sq3 * minv - m3 * m3
    s3 = g3[None] / jnp.sqrt(v3 + 1e-5)
    a3 = W3.T * s3
    c3 = (b3[None] - m3) * s3 + be3[None]

    xt = x.transpose(0, 2, 1)
    mxh, smh, mxs, sms = pl.pallas_call(
        _p4_body,
        grid=grid,
        in_specs=[xspec,
                  pl.BlockSpec((1, 3, NBLK), lambda b_, j: (b_, 0, j)),
                  full((3, 64)), full((1, 64)),
                  full((64, 128)), full((1, 128)),
                  full((128, 256)), full((1, 256)),
                  pl.BlockSpec((1, 1, 32), lambda b_, j: (b_, 0, 0),
                               memory_space=pltpu.SMEM),
                  pl.BlockSpec((1, 1, 2), lambda b_, j: (b_, 0, 0),
                               memory_space=pltpu.SMEM)],
        out_specs=[pl.BlockSpec((1, 1, 256), lambda b_, j: (b_, 0, 0)),
                   pl.BlockSpec((1, 1, 256), lambda b_, j: (b_, 0, 0)),
                   pl.BlockSpec((1, 1, 3), lambda b_, j: (b_, 0, 0)),
                   pl.BlockSpec((1, 1, 3), lambda b_, j: (b_, 0, 0))],
        out_shape=[sds((B, 1, 256)), sds((B, 1, 256)),
                   sds((B, 1, 3)), sds((B, 1, 3))],
    )(x, xt, a1, c1, a2, c2, a3, c3, ctr, prm)

    wpt = Wp.T
    out = pl.pallas_call(
        _p5_body,
        out_shape=sds((B, 512)),
    )(mxh.reshape(B, 256), mxs.reshape(B, 3),
      smh.reshape(B, 256), sms.reshape(B, 3),
      wpt[0:256], wpt[256:259], wpt[259:515], wpt[515:518],
      bp[None], ln_g[None], ln_b[None])
    return out


# speedup vs baseline: 18.1005x; 1.0066x over previous
import functools

import jax
import jax.numpy as jnp
from jax import lax
from jax.experimental import pallas as pl
from jax.experimental.pallas import tpu as pltpu
from jax.experimental.pallas import tpu_sc as plsc

F32 = jnp.float32
NBLK = 16384
N_PTS = 16384
NBATCH = 16


def _p2_body(x_ref, a1_ref, c1_ref, w_ref, b_ref, su_ref, sq_ref):
    bi = pl.program_id(0)
    j = pl.program_id(1)
    xb = x_ref[0]
    h1 = jnp.maximum(
        jnp.dot(xb, a1_ref[...], preferred_element_type=F32) + c1_ref[...], 0.0)
    u = jnp.dot(h1, w_ref[...], preferred_element_type=F32) + b_ref[...]
    su = jnp.sum(u, axis=0, keepdims=True)
    sq = jnp.sum(u * u, axis=0, keepdims=True)
    first = jnp.logical_and(bi == 0, j == 0)

    @pl.when(first)
    def _():
        su_ref[...] = su
        sq_ref[...] = sq

    @pl.when(jnp.logical_not(first))
    def _():
        su_ref[...] += su
        sq_ref[...] += sq


def _p3_body(x_ref, a1_ref, c1_ref, a2_ref, c2_ref, w_ref, b_ref,
             su_ref, sq_ref):
    bi = pl.program_id(0)
    j = pl.program_id(1)
    xb = x_ref[0]
    h1 = jnp.maximum(
        jnp.dot(xb, a1_ref[...], preferred_element_type=F32) + c1_ref[...], 0.0)
    h2 = jnp.maximum(
        jnp.dot(h1, a2_ref[...], preferred_element_type=F32) + c2_ref[...], 0.0)
    u = jnp.dot(h2, w_ref[...], preferred_element_type=F32) + b_ref[...]
    su = jnp.sum(u, axis=0, keepdims=True)
    sq = jnp.sum(u * u, axis=0, keepdims=True)
    first = jnp.logical_and(bi == 0, j == 0)

    @pl.when(first)
    def _():
        su_ref[...] = su
        sq_ref[...] = sq

    @pl.when(jnp.logical_not(first))
    def _():
        su_ref[...] += su
        sq_ref[...] += sq


def _p4_body(x_ref, xt_ref, a1_ref, c1_ref, a2_ref, c2_ref, a3_ref, c3_ref,
             ctr_ref, prm_ref,
             mxh_ref, smh_ref, mxs_ref, sms_ref):
    j = pl.program_id(1)
    xb = x_ref[0]
    h1 = jnp.maximum(
        jnp.dot(xb, a1_ref[...], preferred_element_type=F32) + c1_ref[...], 0.0)
    h2 = jnp.maximum(
        jnp.dot(h1, a2_ref[...], preferred_element_type=F32) + c2_ref[...], 0.0)
    h3 = jnp.maximum(
        jnp.dot(h2, a3_ref[...], preferred_element_type=F32) + c3_ref[...], 0.0)
    mxh = jnp.max(h3, axis=0, keepdims=True)[None]
    smh = jnp.sum(h3, axis=0, keepdims=True)[None]

    xtb = xt_ref[0]
    xs = xtb[0:1]
    yv = xtb[1:2]
    zs = xtb[2:3]
    t = (yv - prm_ref[0, 0, 0]) / prm_ref[0, 0, 1] * 10.0
    bk = t.astype(jnp.int32)
    cx = jnp.zeros((1, NBLK), F32)
    cz = jnp.zeros((1, NBLK), F32)
    for k in range(11):
        mk = bk == k
        cx = jnp.where(mk, ctr_ref[0, 0, k], cx)
        cz = jnp.where(mk, ctr_ref[0, 0, 16 + k], cz)
    dx = xs - cx
    dz = zs - cz
    r = jnp.sqrt(dx * dx + dz * dz)
    inv = jnp.where(r > 0.0, 1.0 / r, 0.0)
    sn = dz * inv
    cs = jnp.where(r > 0.0, dx * inv, 1.0)

    def c3_(a, b, c):
        return jnp.concatenate(
            [jnp.reshape(a, (1, 1, 1)), jnp.reshape(b, (1, 1, 1)),
             jnp.reshape(c, (1, 1, 1))], axis=2)

    mxs = c3_(jnp.max(sn), jnp.max(cs), jnp.max(r))
    sms = c3_(jnp.sum(sn), jnp.sum(cs), jnp.sum(r))

    @pl.when(j == 0)
    def _():
        mxh_ref[...] = mxh
        smh_ref[...] = smh
        mxs_ref[...] = mxs
        sms_ref[...] = sms

    @pl.when(j != 0)
    def _():
        mxh_ref[...] = jnp.maximum(mxh_ref[...], mxh)
        smh_ref[...] += smh
        mxs_ref[...] = jnp.maximum(mxs_ref[...], mxs)
        sms_ref[...] += sms


def _p5_body(mxh_ref, mxs_ref, smh_ref, sms_ref,
             w1_ref, w2_ref, w3_ref, w4_ref,
             bp_ref, g_ref, b_ref, out_ref):
    invn = 1.0 / N_PTS
    o = (jnp.dot(mxh_ref[...], w1_ref[...], preferred_element_type=F32)
         + jnp.dot(mxs_ref[...], w2_ref[...], preferred_element_type=F32)
         + jnp.dot(smh_ref[...] * invn, w3_ref[...], preferred_element_type=F32)
         + jnp.dot(sms_ref[...] * invn, w4_ref[...], preferred_element_type=F32)
         + bp_ref[...])
    mu = jnp.mean(o, axis=1, keepdims=True)
    var = jnp.mean((o - mu) ** 2, axis=1, keepdims=True)
    out_ref[...] = (o - mu) / jnp.sqrt(var + 1e-5) * g_ref[...] + b_ref[...]


def _sc_body(xs_hbm, ys_hbm, zs_hbm, part_hbm, st_hbm, xv, yv, zv, acc, stv):
    c = lax.axis_index("c")
    s = lax.axis_index("s")
    w = s * 2 + c

    @pl.when(w < NBATCH)
    def _():
        base = w * N_PTS
        pltpu.sync_copy(xs_hbm.at[pl.ds(base, N_PTS)], xv)
        pltpu.sync_copy(ys_hbm.at[pl.ds(base, N_PTS)], yv)
        pltpu.sync_copy(zs_hbm.at[pl.ds(base, N_PTS)], zv)
        nit = N_PTS // 16
        big = jnp.float32(3.4e38)
        zero = jnp.zeros((16,), F32)
        init = (jnp.full((16,), big, F32), jnp.full((16,), -big, F32),
                zero, zero, zero, zero, zero, zero, zero, zero, zero)

        def body1(i, cr):
            mn, mx, sx, sy, sz, sxx, sxy, sxz, syy, syz, szz = cr
            off = i * 16
            xw = xv[pl.ds(off, 16)]
            yw = yv[pl.ds(off, 16)]
            zw = zv[pl.ds(off, 16)]
            return (jnp.minimum(mn, yw), jnp.maximum(mx, yw),
                    sx + xw, sy + yw, sz + zw,
                    sxx + xw * xw, sxy + xw * yw, sxz + xw * zw,
                    syy + yw * yw, syz + yw * zw, szz + zw * zw)

        st = lax.fori_loop(0, nit, body1, init)
        for i in range(11):
            stv[i] = st[i]
        stv[11] = zero
        pltpu.sync_copy(stv, st_hbm.at[w])

        mn_s = jnp.min(st[0])
        mx_s = jnp.max(st[1])
        den_s = mx_s - mn_s + 1e-6
        mn_v = jnp.full((16,), mn_s, F32)
        den_v = jnp.full((16,), den_s, F32)

        for i in range(3):
            for rr in range(16):
                acc[i, rr] = zero
        lanes = lax.iota(jnp.int32, 16)
        idx0 = jnp.zeros((16,), jnp.int32)
        idx1 = idx0 + 1
        idx2 = idx0 + 2
        ones_f = jnp.ones((16,), F32)

        def body2(i, carry):
            off = i * 16
            y16 = yv[pl.ds(off, 16)]
            x16 = xv[pl.ds(off, 16)]
            z16 = zv[pl.ds(off, 16)]
            t = (y16 - mn_v) / den_v * 10.0
            bk = t.astype(jnp.int32)
            plsc.addupdate_scatter(acc, [idx0, bk, lanes], ones_f)
            plsc.addupdate_scatter(acc, [idx1, bk, lanes], x16)
            plsc.addupdate_scatter(acc, [idx2, bk, lanes], z16)
            return carry

        lax.fori_loop(0, nit, body2, 0)
        pltpu.sync_copy(acc, part_hbm.at[w])


def _sc_hist(xs_f, ys_f, zs_f):
    fn = functools.partial(
        pl.kernel,
        mesh=plsc.VectorSubcoreMesh(core_axis_name="c", subcore_axis_name="s"),
        out_type=[jax.ShapeDtypeStruct((NBATCH, 3, 16, 16), F32),
                  jax.ShapeDtypeStruct((NBATCH, 12, 16), F32)],
        compiler_params=pltpu.CompilerParams(needs_layout_passes=False),
        scratch_types=[
            pltpu.VMEM((N_PTS,), F32),
            pltpu.VMEM((N_PTS,), F32),
            pltpu.VMEM((N_PTS,), F32),
            pltpu.VMEM((3, 16, 16), F32),
            pltpu.VMEM((12, 16), F32),
        ],
    )(_sc_body)
    return fn(xs_f, ys_f, zs_f)


def kernel(x, W1, b1, g1, be1, W2, b2, g2, be2, W3, b3, g3, be3,
           Wp, bp, ln_g, ln_b):
    B, N, _ = x.shape
    NJ = N // NBLK
    M = B * N
    minv = 1.0 / M
    grid = (B, NJ)
    xspec = pl.BlockSpec((1, NBLK, 3), lambda b_, j: (b_, j, 0))

    def full(shp):
        return pl.BlockSpec(shp, lambda b_, j: (0,) * len(shp))

    def sds(shp):
        return jax.ShapeDtypeStruct(shp, F32)

    xs_f = x[:, :, 0].reshape(M)
    ys_f = x[:, :, 1].reshape(M)
    zs_f = x[:, :, 2].reshape(M)
    part, st = _sc_hist(xs_f, ys_f, zs_f)

    ymn_b = jnp.min(st[:, 0, :], axis=1)
    ymx_b = jnp.max(st[:, 1, :], axis=1)
    den_b = ymx_b - ymn_b + 1e-6
    mom = jnp.sum(st[:, 2:11, :], axis=(0, 2))
    mu3 = mom[0:3] * minv
    cov = (jnp.stack([
        jnp.stack([mom[3], mom[4], mom[5]]),
        jnp.stack([mom[4], mom[6], mom[7]]),
        jnp.stack([mom[5], mom[7], mom[8]]),
    ]) * minv - mu3[:, None] * mu3[None, :])
    m1 = mu3 @ W1.T + b1
    v1 = jnp.einsum("jc,cd,jd->j", W1, cov, W1)
    s1 = g1 / jnp.sqrt(v1 + 1e-5)
    a1 = W1.T * s1[None]
    c1 = ((b1 - m1) * s1 + be1)[None]

    agg = jnp.sum(part, axis=3)
    cnt = agg[:, 0]
    safe = jnp.maximum(cnt, 1.0)
    pos = cnt > 0.0
    cx = jnp.where(pos, agg[:, 1] / safe, 0.0)
    cz = jnp.where(pos, agg[:, 2] / safe, 0.0)
    ctr = jnp.concatenate([cx, cz], axis=1).reshape(B, 1, 32)
    prm = jnp.stack([ymn_b, den_b], axis=1).reshape(B, 1, 2)

    su2, sq2 = pl.pallas_call(
        _p2_body,
        grid=grid,
        in_specs=[xspec, full((3, 64)), full((1, 64)),
                  full((64, 128)), full((1, 128))],
        out_specs=[full((1, 128)), full((1, 128))],
        out_shape=[sds((1, 128)), sds((1, 128))],
    )(x, a1, c1, W2.T, b2[None])

    m2 = su2 * minv
    v2 = sq2 * minv - m2 * m2
    s2 = g2[None] / jnp.sqrt(v2 + 1e-5)
    a2 = W2.T * s2
    c2 = (b2[None] - m2) * s2 + be2[None]

    su3, sq3 = pl.pallas_call(
        _p3_body,
        grid=grid,
        in_specs=[xspec, full((3, 64)), full((1, 64)),
                  full((64, 128)), full((1, 128)),
                  full((128, 256)), full((1, 256))],
        out_specs=[full((1, 256)), full((1, 256))],
        out_shape=[sds((1, 256)), sds((1, 256))],
    )(x, a1, c1, a2, c2, W3.T, b3[None])

    m3 = su3 * minv
    v3 = sq3 * minv - m3 * m3
    s3 = g3[None] / jnp.sqrt(v3 + 1e-5)
    a3 = W3.T * s3
    c3 = (b3[None] - m3) * s3 + be3[None]

    xt = x.transpose(0, 2, 1)
    mxh, smh, mxs, sms = pl.pallas_call(
        _p4_body,
        grid=grid,
        in_specs=[xspec,
                  pl.BlockSpec((1, 3, NBLK), lambda b_, j: (b_, 0, j)),
                  full((3, 64)), full((1, 64)),
                  full((64, 128)), full((1, 128)),
                  full((128, 256)), full((1, 256)),
                  pl.BlockSpec((1, 1, 32), lambda b_, j: (b_, 0, 0),
                               memory_space=pltpu.SMEM),
                  pl.BlockSpec((1, 1, 2), lambda b_, j: (b_, 0, 0),
                               memory_space=pltpu.SMEM)],
        out_specs=[pl.BlockSpec((1, 1, 256), lambda b_, j: (b_, 0, 0)),
                   pl.BlockSpec((1, 1, 256), lambda b_, j: (b_, 0, 0)),
                   pl.BlockSpec((1, 1, 3), lambda b_, j: (b_, 0, 0)),
                   pl.BlockSpec((1, 1, 3), lambda b_, j: (b_, 0, 0))],
        out_shape=[sds((B, 1, 256)), sds((B, 1, 256)),
                   sds((B, 1, 3)), sds((B, 1, 3))],
    )(x, xt, a1, c1, a2, c2, a3, c3, ctr, prm)

    wpt = Wp.T
    out = pl.pallas_call(
        _p5_body,
        out_shape=sds((B, 512)),
    )(mxh.reshape(B, 256), mxs.reshape(B, 3),
      smh.reshape(B, 256), sms.reshape(B, 3),
      wpt[0:256], wpt[256:259], wpt[259:515], wpt[515:518],
      bp[None], ln_g[None], ln_b[None])
    return out
